# Initial kernel scaffold; baseline (speedup 1.0000x reference)
import jax
import jax.numpy as jnp
from jax.experimental import pallas as pl


def kernel(x, adj, W_in, b_in, W1_0, b1_0, W2_0, b2_0, W1_1, b1_1, W2_1, b2_1, W_out, b_out):
    raise NotImplementedError("write your pallas kernel here")


# speedup vs baseline: 2.8545x; 2.8545x over previous
import functools

import jax
import jax.numpy as jnp
from jax import lax
from jax.experimental import pallas as pl
from jax.experimental.pallas import tpu as pltpu
from jax.experimental.pallas import tpu_sc as plsc

NN = 10000
D = 128
E = 320000
NC = 2
NS = 16
GPT = 80
E_PAD = NC * NS * GPT * 128
ACC_ROWS = 10112
ROWS_PER_TILE_ZERO = ACC_ROWS // NS
ROWS_PER_TILE_OUT = 624
CHUNK = 2
IDX_HALF = GPT // 2
N_CHUNKS = IDX_HALF // CHUNK


def _sc_segsum_body(h_hbm, src_hbm, dst_hbm, zeros_hbm, out_hbm,
                    src_v, dst_v, rows_v, acc_sh, sem):
    c = lax.axis_index("c")
    s = lax.axis_index("s")
    tid = c * NS + s

    pltpu.sync_copy(zeros_hbm.at[pl.ds(s * ROWS_PER_TILE_ZERO, ROWS_PER_TILE_ZERO)],
                    acc_sh.at[pl.ds(s * ROWS_PER_TILE_ZERO, ROWS_PER_TILE_ZERO)])
    plsc.subcore_barrier()

    def chunk_body(i, carry):
        base = i * CHUNK
        cps = []
        for j in range(CHUNK):
            cps.append(pltpu.async_copy(h_hbm.at[src_v.at[base + j]],
                                        rows_v.at[j], sem))
        for cp in cps:
            cp.wait()
        for j in range(CHUNK):
            pltpu.sync_copy(rows_v.at[j], acc_sh.at[dst_v.at[base + j]],
                            add=True)
        return carry

    for half in range(2):
        pltpu.sync_copy(src_hbm.at[pl.ds(tid * GPT + half * IDX_HALF, IDX_HALF)],
                        src_v)
        pltpu.sync_copy(dst_hbm.at[pl.ds(tid * GPT + half * IDX_HALF, IDX_HALF)],
                        dst_v)
        lax.fori_loop(0, N_CHUNKS, chunk_body, 0)
    plsc.subcore_barrier()
    pltpu.sync_copy(acc_sh.at[pl.ds(s * ROWS_PER_TILE_OUT, ROWS_PER_TILE_OUT)],
                    out_hbm.at[c, pl.ds(s * ROWS_PER_TILE_OUT, ROWS_PER_TILE_OUT)])

    @pl.when(s == NS - 1)
    def _copy_tail():
        base = NS * ROWS_PER_TILE_OUT
        pltpu.sync_copy(acc_sh.at[pl.ds(base, NN - base)],
                        out_hbm.at[c, pl.ds(base, NN - base)])


@functools.cache
def _sc_segsum():
    return pl.kernel(
        _sc_segsum_body,
        out_type=jax.ShapeDtypeStruct((NC, NN, D), jnp.float32),
        mesh=plsc.VectorSubcoreMesh(core_axis_name="c", subcore_axis_name="s",
                                    num_cores=NC, num_subcores=NS),
        scratch_types=[
            pltpu.VMEM((IDX_HALF, 128), jnp.int32),
            pltpu.VMEM((IDX_HALF, 128), jnp.int32),
            pltpu.VMEM((CHUNK, 128, D), jnp.float32),
            pltpu.VMEM_SHARED((ACC_ROWS, D), jnp.float32),
            pltpu.SemaphoreType.DMA,
        ],
    )


def _linear_body(x_ref, w_ref, b_ref, o_ref):
    o_ref[...] = jnp.dot(x_ref[...], w_ref[...],
                         preferred_element_type=jnp.float32) + b_ref[...]


_linear = pl.pallas_call(
    _linear_body,
    out_shape=jax.ShapeDtypeStruct((NN, D), jnp.float32),
)


def _gin_mlp_body(h_ref, ma_ref, mb_ref, w1_ref, b1_ref, w2_ref, b2_ref, o_ref):
    z = h_ref[...] + ma_ref[...] + mb_ref[...]
    t = jnp.maximum(jnp.dot(z, w1_ref[...], preferred_element_type=jnp.float32)
                    + b1_ref[...], 0.0)
    o_ref[...] = jnp.dot(t, w2_ref[...],
                         preferred_element_type=jnp.float32) + b2_ref[...]


_gin_mlp = pl.pallas_call(
    _gin_mlp_body,
    out_shape=jax.ShapeDtypeStruct((NN, D), jnp.float32),
)


def _final_body(h0_ref, h1_ref, ma_ref, mb_ref, w1_ref, b1_ref, w2_ref, b2_ref,
                wo0_ref, wo1_ref, wo2_ref, bo_ref, o_ref):
    z = h1_ref[...] + ma_ref[...] + mb_ref[...]
    t = jnp.maximum(jnp.dot(z, w1_ref[...], preferred_element_type=jnp.float32)
                    + b1_ref[...], 0.0)
    h2 = jnp.dot(t, w2_ref[...], preferred_element_type=jnp.float32) + b2_ref[...]
    o_ref[...] = (jnp.dot(h0_ref[...], wo0_ref[...], preferred_element_type=jnp.float32)
                  + jnp.dot(h1_ref[...], wo1_ref[...], preferred_element_type=jnp.float32)
                  + jnp.dot(h2, wo2_ref[...], preferred_element_type=jnp.float32)
                  + bo_ref[...])


_final = pl.pallas_call(
    _final_body,
    out_shape=jax.ShapeDtypeStruct((NN, D), jnp.float32),
)


def kernel(x, adj, W_in, b_in, W1_0, b1_0, W2_0, b2_0,
           W1_1, b1_1, W2_1, b2_1, W_out, b_out):
    src = adj[0]
    dst = adj[1]
    src_p = jnp.concatenate(
        [src, jnp.zeros((E_PAD - E,), jnp.int32)]).reshape(-1, 128)
    dst_p = jnp.concatenate(
        [dst, jnp.full((E_PAD - E,), NN, jnp.int32)]).reshape(-1, 128)
    zeros_acc = jnp.zeros((ACC_ROWS, D), jnp.float32)

    b_in2 = b_in.reshape(1, D)
    b1_02 = b1_0.reshape(1, D)
    b2_02 = b2_0.reshape(1, D)
    b1_12 = b1_1.reshape(1, D)
    b2_12 = b2_1.reshape(1, D)
    b_out2 = b_out.reshape(1, D)
    wo0 = W_out[0:D]
    wo1 = W_out[D:2 * D]
    wo2 = W_out[2 * D:3 * D]

    segsum = _sc_segsum()
    h0 = _linear(x, W_in, b_in2)
    msg1 = segsum(h0, src_p, dst_p, zeros_acc)
    h1 = _gin_mlp(h0, msg1[0], msg1[1], W1_0, b1_02, W2_0, b2_02)
    msg2 = segsum(h1, src_p, dst_p, zeros_acc)
    out = _final(h0, h1, msg2[0], msg2[1], W1_1, b1_12, W2_1, b2_12,
                 wo0, wo1, wo2, b_out2)
    return out


# speedup vs baseline: 3.1224x; 1.0939x over previous
import functools

import jax
import jax.numpy as jnp
from jax import lax
from jax.experimental import pallas as pl
from jax.experimental.pallas import tpu as pltpu
from jax.experimental.pallas import tpu_sc as plsc

NN = 10000
D = 128
E = 320000
NC = 2
NS = 16
GPT = 80
E_PAD = NC * NS * GPT * 128
ACC_ROWS = 10112
ROWS_PER_TILE_ZERO = ACC_ROWS // NS
ROWS_PER_TILE_OUT = 624
CHUNK = 2
IDX_HALF = GPT // 2
N_CHUNKS = IDX_HALF // CHUNK


def _sc_segsum_body(h_hbm, src_hbm, dst_hbm, zeros_hbm, out_hbm,
                    src_v, dst_v, rows_v, acc_sh, sems):
    c = lax.axis_index("c")
    s = lax.axis_index("s")
    tid = c * NS + s

    pltpu.sync_copy(zeros_hbm.at[pl.ds(s * ROWS_PER_TILE_ZERO, ROWS_PER_TILE_ZERO)],
                    acc_sh.at[pl.ds(s * ROWS_PER_TILE_ZERO, ROWS_PER_TILE_ZERO)])
    plsc.subcore_barrier()

    for half in range(2):
        pltpu.sync_copy(src_hbm.at[pl.ds(tid * GPT + half * IDX_HALF, IDX_HALF)],
                        src_v)
        pltpu.sync_copy(dst_hbm.at[pl.ds(tid * GPT + half * IDX_HALF, IDX_HALF)],
                        dst_v)

        for b in range(CHUNK):
            pltpu.async_copy(h_hbm.at[src_v.at[b]], rows_v.at[b], sems.at[b])

        def steady(i, carry):
            for b in range(CHUNK):
                g = i * CHUNK + b
                pltpu.make_async_copy(h_hbm.at[src_v.at[g]], rows_v.at[b],
                                      sems.at[b]).wait()
                pltpu.sync_copy(rows_v.at[b], acc_sh.at[dst_v.at[g]],
                                add=True)
                pltpu.async_copy(h_hbm.at[src_v.at[g + CHUNK]], rows_v.at[b],
                                 sems.at[b])
            return carry

        lax.fori_loop(0, N_CHUNKS - 1, steady, 0)
        for b in range(CHUNK):
            g = IDX_HALF - CHUNK + b
            pltpu.make_async_copy(h_hbm.at[src_v.at[g]], rows_v.at[b],
                                  sems.at[b]).wait()
            pltpu.sync_copy(rows_v.at[b], acc_sh.at[dst_v.at[g]], add=True)
    plsc.subcore_barrier()
    pltpu.sync_copy(acc_sh.at[pl.ds(s * ROWS_PER_TILE_OUT, ROWS_PER_TILE_OUT)],
                    out_hbm.at[c, pl.ds(s * ROWS_PER_TILE_OUT, ROWS_PER_TILE_OUT)])

    @pl.when(s == NS - 1)
    def _copy_tail():
        base = NS * ROWS_PER_TILE_OUT
        pltpu.sync_copy(acc_sh.at[pl.ds(base, NN - base)],
                        out_hbm.at[c, pl.ds(base, NN - base)])


@functools.cache
def _sc_segsum():
    return pl.kernel(
        _sc_segsum_body,
        out_type=jax.ShapeDtypeStruct((NC, NN, D), jnp.float32),
        mesh=plsc.VectorSubcoreMesh(core_axis_name="c", subcore_axis_name="s",
                                    num_cores=NC, num_subcores=NS),
        scratch_types=[
            pltpu.VMEM((IDX_HALF, 128), jnp.int32),
            pltpu.VMEM((IDX_HALF, 128), jnp.int32),
            pltpu.VMEM((CHUNK, 128, D), jnp.float32),
            pltpu.VMEM_SHARED((ACC_ROWS, D), jnp.float32),
            pltpu.SemaphoreType.DMA((CHUNK,)),
        ],
    )


def _linear_body(x_ref, w_ref, b_ref, o_ref):
    o_ref[...] = jnp.dot(x_ref[...], w_ref[...],
                         preferred_element_type=jnp.float32) + b_ref[...]


_linear = pl.pallas_call(
    _linear_body,
    out_shape=jax.ShapeDtypeStruct((NN, D), jnp.float32),
)


def _gin_mlp_body(h_ref, ma_ref, mb_ref, w1_ref, b1_ref, w2_ref, b2_ref, o_ref):
    z = h_ref[...] + ma_ref[...] + mb_ref[...]
    t = jnp.maximum(jnp.dot(z, w1_ref[...], preferred_element_type=jnp.float32)
                    + b1_ref[...], 0.0)
    o_ref[...] = jnp.dot(t, w2_ref[...],
                         preferred_element_type=jnp.float32) + b2_ref[...]


_gin_mlp = pl.pallas_call(
    _gin_mlp_body,
    out_shape=jax.ShapeDtypeStruct((NN, D), jnp.float32),
)


def _final_body(h0_ref, h1_ref, ma_ref, mb_ref, w1_ref, b1_ref, w2_ref, b2_ref,
                wo0_ref, wo1_ref, wo2_ref, bo_ref, o_ref):
    z = h1_ref[...] + ma_ref[...] + mb_ref[...]
    t = jnp.maximum(jnp.dot(z, w1_ref[...], preferred_element_type=jnp.float32)
                    + b1_ref[...], 0.0)
    h2 = jnp.dot(t, w2_ref[...], preferred_element_type=jnp.float32) + b2_ref[...]
    o_ref[...] = (jnp.dot(h0_ref[...], wo0_ref[...], preferred_element_type=jnp.float32)
                  + jnp.dot(h1_ref[...], wo1_ref[...], preferred_element_type=jnp.float32)
                  + jnp.dot(h2, wo2_ref[...], preferred_element_type=jnp.float32)
                  + bo_ref[...])


_final = pl.pallas_call(
    _final_body,
    out_shape=jax.ShapeDtypeStruct((NN, D), jnp.float32),
)


def kernel(x, adj, W_in, b_in, W1_0, b1_0, W2_0, b2_0,
           W1_1, b1_1, W2_1, b2_1, W_out, b_out):
    src = adj[0]
    dst = adj[1]
    src_p = jnp.concatenate(
        [src, jnp.zeros((E_PAD - E,), jnp.int32)]).reshape(-1, 128)
    dst_p = jnp.concatenate(
        [dst, jnp.full((E_PAD - E,), NN, jnp.int32)]).reshape(-1, 128)
    zeros_acc = jnp.zeros((ACC_ROWS, D), jnp.float32)

    b_in2 = b_in.reshape(1, D)
    b1_02 = b1_0.reshape(1, D)
    b2_02 = b2_0.reshape(1, D)
    b1_12 = b1_1.reshape(1, D)
    b2_12 = b2_1.reshape(1, D)
    b_out2 = b_out.reshape(1, D)
    wo0 = W_out[0:D]
    wo1 = W_out[D:2 * D]
    wo2 = W_out[2 * D:3 * D]

    segsum = _sc_segsum()
    h0 = _linear(x, W_in, b_in2)
    msg1 = segsum(h0, src_p, dst_p, zeros_acc)
    h1 = _gin_mlp(h0, msg1[0], msg1[1], W1_0, b1_02, W2_0, b2_02)
    msg2 = segsum(h1, src_p, dst_p, zeros_acc)
    out = _final(h0, h1, msg2[0], msg2[1], W1_1, b1_12, W2_1, b2_12,
                 wo0, wo1, wo2, b_out2)
    return out
